# baseline (speedup 1.0000x reference)
import jax
import jax.numpy as jnp
from jax.experimental import pallas as pl

_NT = (((1,), (1,)), ((), ()))
_TN = (((0,), (0,)), ((), ()))


def _gat3_kernel(x_ref, adj_ref, w1_ref, as1_ref, ad1_ref, b1_ref,
                 w2_ref, as2_ref, ad2_ref, b2_ref,
                 w3_ref, as3_ref, ad3_ref, b3_ref, out_ref):
    B, N, in_dim = x_ref.shape

    mbias = [jnp.where(adj_ref[b] > 0.0, 0.0, jnp.float32(-1e30))
             for b in range(B)]

    def layer(xf, w_ref, as_ref, ad_ref, b_ref, store=None):
        w = w_ref[...]
        aa = jnp.concatenate([as_ref[...].reshape(1, -1),
                              ad_ref[...].reshape(1, -1)], axis=0)
        bias = b_ref[...].reshape(1, -1)
        h = jax.lax.dot_general(xf, w, _NT,
                                preferred_element_type=jnp.float32)
        al = jax.lax.dot_general(aa, h, _NT,
                                 preferred_element_type=jnp.float32)
        outs = []
        for b in range(B):
            hb = h[b * N:(b + 1) * N]
            as_col = al[0:1, b * N:(b + 1) * N].T
            ad_row = al[1:2, b * N:(b + 1) * N]
            e = as_col + ad_row
            e = jnp.maximum(e, 0.2 * e)
            e = e + mbias[b]
            p = jnp.exp(e)
            denom = jnp.sum(p, axis=0, keepdims=True)
            a = p * (1.0 / (denom + 1e-16))
            ob = jax.lax.dot_general(a, hb, _TN,
                                     preferred_element_type=jnp.float32)
            if store is None:
                outs.append(ob + bias)
            else:
                store(b, ob + bias)
        if store is None:
            return jnp.concatenate(outs, axis=0)

    def store_out(b, val):
        out_ref[b] = val

    xf = x_ref[...].reshape(B * N, in_dim)
    xf = layer(xf, w1_ref, as1_ref, ad1_ref, b1_ref)
    xf = layer(xf, w2_ref, as2_ref, ad2_ref, b2_ref)
    layer(xf, w3_ref, as3_ref, ad3_ref, b3_ref, store=store_out)


def kernel(batch_graph, adj, W1, a_src1, a_dst1, b1, W2, a_src2, a_dst2, b2,
           W3, a_src3, a_dst3, b3):
    B, N, in_dim = batch_graph.shape
    hid = W1.shape[0]
    out_dim = W3.shape[0]
    BBLK = 2

    def full(shape):
        ndim = len(shape)
        return pl.BlockSpec(shape, lambda g, _n=ndim: (0,) * _n)

    return pl.pallas_call(
        _gat3_kernel,
        grid=(B // BBLK,),
        in_specs=[
            pl.BlockSpec((BBLK, N, in_dim), lambda g: (g, 0, 0)),
            pl.BlockSpec((BBLK, N, N), lambda g: (g, 0, 0)),
            full((hid, in_dim)), full((hid,)), full((hid,)), full((hid,)),
            full((hid, hid)), full((hid,)), full((hid,)), full((hid,)),
            full((out_dim, hid)), full((out_dim,)), full((out_dim,)),
            full((out_dim,)),
        ],
        out_specs=pl.BlockSpec((BBLK, N, out_dim), lambda g: (g, 0, 0)),
        out_shape=jax.ShapeDtypeStruct((B, N, out_dim), jnp.float32),
    )(batch_graph, adj,
      W1, a_src1, a_dst1, b1,
      W2, a_src2, a_dst2, b2,
      W3, a_src3, a_dst3, b3)


# speedup vs baseline: 1.1963x; 1.1963x over previous
import jax
import jax.numpy as jnp
from jax.experimental import pallas as pl

_NT = (((1,), (1,)), ((), ()))
_TN = (((0,), (0,)), ((), ()))


def _gat3_kernel(x_ref, adj_ref, w1_ref, as1_ref, ad1_ref, b1_ref,
                 w2_ref, as2_ref, ad2_ref, b2_ref,
                 w3_ref, as3_ref, ad3_ref, b3_ref, out_ref):
    B, N, in_dim = x_ref.shape

    mbias = [jnp.where(adj_ref[b] > 0.0, 0.0, jnp.float32(-1e30))
             for b in range(B)]

    def layer(xf, w_ref, as_ref, ad_ref, b_ref, store=None):
        w = w_ref[...]
        aa = jnp.concatenate([as_ref[...].reshape(1, -1),
                              ad_ref[...].reshape(1, -1)], axis=0)
        bias = b_ref[...].reshape(1, -1)
        h = jax.lax.dot_general(xf, w, _NT,
                                preferred_element_type=jnp.float32)
        al = jax.lax.dot_general(aa, h, _NT,
                                 preferred_element_type=jnp.float32)
        outs = []
        for b in range(B):
            hb = h[b * N:(b + 1) * N]
            as_col = al[0:1, b * N:(b + 1) * N].T
            ad_row = al[1:2, b * N:(b + 1) * N]
            e = as_col + ad_row
            e = jnp.maximum(e, 0.2 * e)
            e = e + mbias[b]
            p = jnp.exp(e)
            denom = jnp.sum(p, axis=0, keepdims=True)
            a = p * (1.0 / (denom + 1e-16))
            ob = jax.lax.dot_general(a, hb, _TN,
                                     preferred_element_type=jnp.float32)
            if store is None:
                outs.append(ob + bias)
            else:
                store(b, ob + bias)
        if store is None:
            return jnp.concatenate(outs, axis=0)

    def store_out(b, val):
        out_ref[b] = val

    xf = x_ref[...].reshape(B * N, in_dim)
    xf = layer(xf, w1_ref, as1_ref, ad1_ref, b1_ref)
    xf = layer(xf, w2_ref, as2_ref, ad2_ref, b2_ref)
    layer(xf, w3_ref, as3_ref, ad3_ref, b3_ref, store=store_out)


def kernel(batch_graph, adj, W1, a_src1, a_dst1, b1, W2, a_src2, a_dst2, b2,
           W3, a_src3, a_dst3, b3):
    B, N, _ = batch_graph.shape
    out_dim = W3.shape[0]
    return pl.pallas_call(
        _gat3_kernel,
        out_shape=jax.ShapeDtypeStruct((B, N, out_dim), jnp.float32),
    )(batch_graph, adj,
      W1, a_src1, a_dst1, b1,
      W2, a_src2, a_dst2, b2,
      W3, a_src3, a_dst3, b3)
